# baseline (speedup 1.0000x reference)
import functools

import jax
import jax.numpy as jnp
from jax import lax
from jax.experimental import pallas as pl
from jax.experimental.pallas import tpu as pltpu
from jax.experimental.pallas import tpu_sc as plsc

BATCH = 4096
SEQ_LEN = 200
DIM = 16

_INFO = plsc.get_sparse_core_info()
NC, NS = _INFO.num_cores, _INFO.num_subcores
NW = NC * NS

N = BATCH * SEQ_LEN
C = 128
N_PER_W = N // NW
ROWS_PER_W = N_PER_W // C
K = 8
CK = C * K
GROUPS = ROWS_PER_W // K


def _gather_body(ids_hbm, table_hbm, out_hbm, idx_v, rows_v, sem):
    wid = lax.axis_index("s") * NC + lax.axis_index("c")
    pltpu.sync_copy(ids_hbm.at[pl.ds(wid * ROWS_PER_W, ROWS_PER_W)], idx_v)

    @pl.loop(0, GROUPS)
    def _group(g):
        descs = [
            pltpu.async_copy(
                table_hbm.at[idx_v.at[g * K + j]],
                rows_v.at[pl.ds(j * C, C)],
                sem,
            )
            for j in range(K)
        ]
        for d in descs:
            d.wait()
        pltpu.sync_copy(rows_v, out_hbm.at[pl.ds(wid * N_PER_W + g * CK, CK)])


@jax.jit
def _embedding_gather(ids_flat_2d, table):
    mesh = plsc.VectorSubcoreMesh(core_axis_name="c", subcore_axis_name="s")
    return pl.kernel(
        _gather_body,
        out_type=jax.ShapeDtypeStruct((N, DIM), jnp.float32),
        mesh=mesh,
        scratch_types=[
            pltpu.VMEM((ROWS_PER_W, C), jnp.int32),
            pltpu.VMEM((CK, DIM), jnp.float32),
            pltpu.SemaphoreType.DMA,
        ],
        compiler_params=pltpu.CompilerParams(use_tc_tiling_on_sc=False),
    )(ids_flat_2d, table)


def kernel(residue_ids, res_id_emb_weight):
    ids_flat_2d = residue_ids.astype(jnp.int32).reshape(N // C, C)
    out = _embedding_gather(ids_flat_2d, res_id_emb_weight)
    return out.reshape(BATCH, SEQ_LEN, DIM)
